# Initial kernel scaffold; baseline (speedup 1.0000x reference)
import jax
import jax.numpy as jnp
from jax.experimental import pallas as pl


def kernel(x, token_table, pos_table):
    raise NotImplementedError("write your pallas kernel here")


# speedup vs baseline: 1.0758x; 1.0758x over previous
import functools

import jax
import jax.numpy as jnp
from jax import lax
from jax.experimental import pallas as pl
from jax.experimental.pallas import tpu as pltpu
from jax.experimental.pallas import tpu_sc as plsc

_LANES = 16
_IDXW = 128
_NC = 2
_NS = 16


@functools.lru_cache(maxsize=None)
def _build(V, D, B, S):
    N = B * S
    NW = _NC * _NS
    BPW = N // NW
    KJ = BPW // _IDXW
    LP = D // _LANES

    assert N % NW == 0 and BPW % _IDXW == 0 and D % _LANES == 0
    assert S % BPW == 0

    mesh = plsc.VectorSubcoreMesh(core_axis_name="c", subcore_axis_name="s")

    @functools.partial(
        pl.kernel,
        mesh=mesh,
        out_type=jax.ShapeDtypeStruct((N, D), jnp.float32),
        scratch_types=[
            pltpu.VMEM((KJ, _IDXW), jnp.int32),
            pltpu.VMEM((BPW, D), jnp.float32),
            pltpu.VMEM((BPW, D), jnp.float32),
            pltpu.SemaphoreType.DMA,
            pltpu.SemaphoreType.DMA,
        ],
    )
    def embed(idx_hbm, tok_hbm, pos_hbm, out_hbm, idx_v, rows_v, pos_v,
              gsem, psem):
        wid = lax.axis_index("s") * _NC + lax.axis_index("c")
        base = wid * BPW
        sbase = lax.rem(base, S)

        pltpu.sync_copy(idx_hbm.at[pl.ds(wid * KJ, KJ)], idx_v)
        pcopy = pltpu.async_copy(pos_hbm.at[pl.ds(sbase, BPW)], pos_v, psem)
        gathers = [
            pltpu.async_copy(tok_hbm.at[idx_v.at[j]],
                             rows_v.at[pl.ds(j * _IDXW, _IDXW)], gsem)
            for j in range(KJ)
        ]
        for g in gathers:
            g.wait()
        pcopy.wait()

        def row_add(i, carry):
            for j in range(LP):
                sl = pl.ds(j * _LANES, _LANES)
                rows_v[i, sl] = rows_v[i, sl] + pos_v[i, sl]
            return carry

        lax.fori_loop(0, BPW, row_add, 0)
        pltpu.sync_copy(rows_v, out_hbm.at[pl.ds(base, BPW)])

    return embed


def kernel(x, token_table, pos_table):
    B, S = x.shape
    V, D = token_table.shape
    idx = x.reshape(-1).astype(jnp.int32).reshape(-1, _IDXW)
    out = _build(V, D, B, S)(idx, token_table, pos_table)
    return out.reshape(B, S, D)


# speedup vs baseline: 1.1367x; 1.0566x over previous
import functools

import jax
import jax.numpy as jnp
from jax import lax
from jax.experimental import pallas as pl
from jax.experimental.pallas import tpu as pltpu
from jax.experimental.pallas import tpu_sc as plsc

_LANES = 16
_IDXW = 128
_NC = 2
_NS = 16


@functools.lru_cache(maxsize=None)
def _build(V, D, B, S):
    N = B * S
    NW = _NC * _NS
    BPW = N // NW
    KJ = BPW // _IDXW
    LP = D // _LANES

    assert N % NW == 0 and BPW % _IDXW == 0 and D % _LANES == 0
    assert S % BPW == 0

    mesh = plsc.VectorSubcoreMesh(core_axis_name="c", subcore_axis_name="s")

    @functools.partial(
        pl.kernel,
        mesh=mesh,
        out_type=jax.ShapeDtypeStruct((N, D), jnp.float32),
        scratch_types=[
            pltpu.VMEM((KJ, _IDXW), jnp.int32),
            pltpu.VMEM((BPW, D), jnp.float32),
            pltpu.VMEM((BPW, D), jnp.float32),
            pltpu.SemaphoreType.DMA,
            pltpu.SemaphoreType.DMA,
        ],
    )
    def embed(idx_hbm, tok_hbm, pos_hbm, out_hbm, idx_v, rows_v, pos_v,
              gsem, psem):
        wid = lax.axis_index("s") * _NC + lax.axis_index("c")
        base = wid * BPW
        sbase = lax.rem(base, S)

        pltpu.sync_copy(idx_hbm.at[pl.ds(wid * KJ, KJ)], idx_v)
        pcopy = pltpu.async_copy(pos_hbm.at[pl.ds(sbase, BPW)], rows_v, psem)
        pcopy.wait()
        gathers = [
            pltpu.async_copy(tok_hbm.at[idx_v.at[j]],
                             rows_v.at[pl.ds(j * _IDXW, _IDXW)], gsem,
                             add=True)
            for j in range(KJ)
        ]
        for g in gathers:
            g.wait()
        pltpu.sync_copy(rows_v, out_hbm.at[pl.ds(base, BPW)])

    return embed


def kernel(x, token_table, pos_table):
    B, S = x.shape
    V, D = token_table.shape
    idx = x.reshape(-1).astype(jnp.int32).reshape(-1, _IDXW)
    out = _build(V, D, B, S)(idx, token_table, pos_table)
    return out.reshape(B, S, D)


# speedup vs baseline: 1.1614x; 1.0218x over previous
import functools

import jax
import jax.numpy as jnp
from jax import lax
from jax.experimental import pallas as pl
from jax.experimental.pallas import tpu as pltpu
from jax.experimental.pallas import tpu_sc as plsc

_LANES = 16
_IDXW = 128
_NC = 2
_NS = 16


@functools.lru_cache(maxsize=None)
def _build(V, D, B, S):
    N = B * S
    NW = _NC * _NS
    BPW = N // NW
    KJ = BPW // _IDXW
    LP = D // _LANES

    assert N % NW == 0 and BPW % _IDXW == 0 and D % _LANES == 0
    assert S % BPW == 0

    mesh = plsc.VectorSubcoreMesh(core_axis_name="c", subcore_axis_name="s")

    @functools.partial(
        pl.kernel,
        mesh=mesh,
        out_type=jax.ShapeDtypeStruct((N, D), jnp.float32),
        scratch_types=(
            [pltpu.VMEM((KJ, _IDXW), jnp.int32),
             pltpu.VMEM((BPW, D), jnp.float32)]
            + [pltpu.SemaphoreType.DMA] * (2 * KJ + 1)
        ),
    )
    def embed(idx_hbm, tok_hbm, pos_hbm, out_hbm, idx_v, rows_v, *sems):
        psems, gsems, ssem = sems[:KJ], sems[KJ:2 * KJ], sems[2 * KJ]
        wid = lax.axis_index("s") * _NC + lax.axis_index("c")
        base = wid * BPW
        sbase = lax.rem(base, S)

        prefills = [
            pltpu.async_copy(pos_hbm.at[pl.ds(sbase + j * _IDXW, _IDXW)],
                             rows_v.at[pl.ds(j * _IDXW, _IDXW)], psems[j])
            for j in range(KJ)
        ]
        pltpu.sync_copy(idx_hbm.at[pl.ds(wid * KJ, KJ)], idx_v)
        gathers = []
        for j in range(KJ):
            prefills[j].wait()
            gathers.append(
                pltpu.async_copy(tok_hbm.at[idx_v.at[j]],
                                 rows_v.at[pl.ds(j * _IDXW, _IDXW)], gsems[j],
                                 add=True))
        stores = []
        for j in range(KJ):
            gathers[j].wait()
            stores.append(
                pltpu.async_copy(rows_v.at[pl.ds(j * _IDXW, _IDXW)],
                                 out_hbm.at[pl.ds(base + j * _IDXW, _IDXW)],
                                 ssem))
        for st in stores:
            st.wait()

    return embed


def kernel(x, token_table, pos_table):
    B, S = x.shape
    V, D = token_table.shape
    idx = x.reshape(-1).astype(jnp.int32).reshape(-1, _IDXW)
    out = _build(V, D, B, S)(idx, token_table, pos_table)
    return out.reshape(B, S, D)


# speedup vs baseline: 1.1641x; 1.0023x over previous
import functools

import jax
import jax.numpy as jnp
from jax import lax
from jax.experimental import pallas as pl
from jax.experimental.pallas import tpu as pltpu
from jax.experimental.pallas import tpu_sc as plsc

_LANES = 16
_IDXW = 64
_NC = 2
_NS = 16


@functools.lru_cache(maxsize=None)
def _build(V, D, B, S):
    N = B * S
    NW = _NC * _NS
    BPW = N // NW
    KJ = BPW // _IDXW
    LP = D // _LANES

    assert N % NW == 0 and BPW % _IDXW == 0 and D % _LANES == 0
    assert S % BPW == 0

    mesh = plsc.VectorSubcoreMesh(core_axis_name="c", subcore_axis_name="s")

    @functools.partial(
        pl.kernel,
        mesh=mesh,
        out_type=jax.ShapeDtypeStruct((N, D), jnp.float32),
        scratch_types=(
            [pltpu.VMEM((KJ, _IDXW), jnp.int32),
             pltpu.VMEM((BPW, D), jnp.float32)]
            + [pltpu.SemaphoreType.DMA] * (2 * KJ + 1)
        ),
    )
    def embed(idx_hbm, tok_hbm, pos_hbm, out_hbm, idx_v, rows_v, *sems):
        psems, gsems, ssem = sems[:KJ], sems[KJ:2 * KJ], sems[2 * KJ]
        wid = lax.axis_index("s") * _NC + lax.axis_index("c")
        base = wid * BPW
        sbase = lax.rem(base, S)

        prefills = [
            pltpu.async_copy(pos_hbm.at[pl.ds(sbase + j * _IDXW, _IDXW)],
                             rows_v.at[pl.ds(j * _IDXW, _IDXW)], psems[j])
            for j in range(KJ)
        ]
        pltpu.sync_copy(idx_hbm.at[pl.ds(wid * KJ, KJ)], idx_v)
        gathers = []
        for j in range(KJ):
            prefills[j].wait()
            gathers.append(
                pltpu.async_copy(tok_hbm.at[idx_v.at[j]],
                                 rows_v.at[pl.ds(j * _IDXW, _IDXW)], gsems[j],
                                 add=True))
        stores = []
        for j in range(KJ):
            gathers[j].wait()
            stores.append(
                pltpu.async_copy(rows_v.at[pl.ds(j * _IDXW, _IDXW)],
                                 out_hbm.at[pl.ds(base + j * _IDXW, _IDXW)],
                                 ssem))
        for st in stores:
            st.wait()

    return embed


def kernel(x, token_table, pos_table):
    B, S = x.shape
    V, D = token_table.shape
    idx = x.reshape(-1).astype(jnp.int32).reshape(-1, _IDXW)
    out = _build(V, D, B, S)(idx, token_table, pos_table)
    return out.reshape(B, S, D)


# speedup vs baseline: 1.1734x; 1.0080x over previous
import functools

import jax
import jax.numpy as jnp
from jax import lax
from jax.experimental import pallas as pl
from jax.experimental.pallas import tpu as pltpu
from jax.experimental.pallas import tpu_sc as plsc

_CH = 64
_NC = 2
_NS = 16


@functools.lru_cache(maxsize=None)
def _build(V, D, B, S):
    N = B * S
    NW = _NC * _NS
    BPW = N // NW
    KJ = BPW // _CH
    WPB = S // BPW

    assert N % NW == 0 and BPW % _CH == 0 and S % BPW == 0

    mesh = plsc.VectorSubcoreMesh(core_axis_name="c", subcore_axis_name="s")

    @functools.partial(
        pl.kernel,
        mesh=mesh,
        out_type=jax.ShapeDtypeStruct((B, S, D), jnp.float32),
        scratch_types=(
            [pltpu.VMEM((BPW,), jnp.int32),
             pltpu.VMEM((BPW, D), jnp.float32)]
            + [pltpu.SemaphoreType.DMA] * (KJ + 2)
        ),
    )
    def embed(idx_hbm, tok_hbm, pos_hbm, out_hbm, idx_v, rows_v, *sems):
        psems, isem, ssem = sems[:KJ], sems[KJ], sems[KJ + 1]
        wid = lax.axis_index("s") * _NC + lax.axis_index("c")
        b = wid // WPB
        sbase = lax.rem(wid, WPB) * BPW

        icopy = pltpu.async_copy(idx_hbm.at[b, pl.ds(sbase, BPW)], idx_v,
                                 isem)
        prefills = [
            pltpu.async_copy(pos_hbm.at[pl.ds(sbase + j * _CH, _CH)],
                             rows_v.at[pl.ds(j * _CH, _CH)], psems[j])
            for j in range(KJ)
        ]
        icopy.wait()
        gathers = []
        for j in range(KJ):
            prefills[j].wait()
            gathers.append(
                pltpu.async_copy(tok_hbm.at[idx_v.at[pl.ds(j * _CH, _CH)]],
                                 rows_v.at[pl.ds(j * _CH, _CH)], psems[j],
                                 add=True))
        stores = []
        for j in range(KJ):
            gathers[j].wait()
            stores.append(
                pltpu.async_copy(rows_v.at[pl.ds(j * _CH, _CH)],
                                 out_hbm.at[b, pl.ds(sbase + j * _CH, _CH)],
                                 ssem))
        for st in stores:
            st.wait()

    return embed


def kernel(x, token_table, pos_table):
    B, S = x.shape
    V, D = token_table.shape
    return _build(V, D, B, S)(x.astype(jnp.int32), token_table, pos_table)
